# speedup vs baseline: 1.9588x; 1.9588x over previous
import functools

import jax
import jax.numpy as jnp
from jax import lax
from jax.experimental import pallas as pl
from jax.experimental.pallas import tpu as pltpu

N = 100000
F = 200000
D = 3
ITERS = 5

F_PAD = 200192
N_PAD = 100352
BF = 512
BN = 512


def _lse2(a, b):
    m = jnp.maximum(a, b)
    return m + jnp.log1p(jnp.exp(-jnp.abs(a - b)))


def _factor_messages(fp, v2f):
    fb = [fp[s] + v2f[2 * 0 + (s >> 2)] + v2f[2 * 1 + ((s >> 1) & 1)]
          + v2f[2 * 2 + (s & 1)] for s in range(8)]
    out = []
    for d in range(3):
        msgs = []
        for x in range(2):
            rows = [fb[s] for s in range(8) if ((s >> (2 - d)) & 1) == x]
            marg = _lse2(_lse2(rows[0], rows[1]), _lse2(rows[2], rows[3]))
            msgs.append(marg - v2f[2 * d + x])
        l = _lse2(msgs[0], msgs[1])
        out.append(msgs[0] - l)
        out.append(msgs[1] - l)
    return out


def _transform(g, f2v, w00, w01, w10, w11):
    out = []
    for d in range(3):
        u0 = g[2 * d] - f2v[2 * d]
        u1 = g[2 * d + 1] - f2v[2 * d + 1]
        v0 = u0 + u0 * w00 + u1 * w10
        v1 = u1 + u0 * w01 + u1 * w11
        l = _lse2(v0, v1)
        out.append(v0 - l)
        out.append(v1 - l)
    return out


def _ka0_body(fp_ref, f2v_ref):
    fp = fp_ref[:, :]
    zero = jnp.zeros_like(fp[0])
    v2f = [zero] * 6
    f2v = _factor_messages(fp, v2f)
    f2v_ref[:, :] = jnp.stack(f2v + [zero, zero])


def _ka_body(fp_ref, g_ref, f2vp_ref, w_ref, f2v_ref):
    fp = fp_ref[:, :]
    g = [g_ref[r, :] for r in range(6)]
    f2vp = [f2vp_ref[r, :] for r in range(6)]
    v2f = _transform(g, f2vp, w_ref[0, 0], w_ref[0, 1], w_ref[1, 0], w_ref[1, 1])
    f2v = _factor_messages(fp, v2f)
    zero = jnp.zeros_like(fp[0])
    f2v_ref[:, :] = jnp.stack(f2v + [zero, zero])


def _kc_body(vbp_ref, vbn_ref):
    vb0 = vbp_ref[0, :] + vbp_ref[2, :]
    vb1 = vbp_ref[1, :] + vbp_ref[3, :]
    l = _lse2(vb0, vb1)
    vbn_ref[:, :] = jnp.stack([vb0 - l, vb1 - l])


def _kf_body(fp_ref, g_ref, f2vp_ref, w_ref, out_ref):
    fp = fp_ref[:, :]
    g = [g_ref[r, :] for r in range(6)]
    f2vp = [f2vp_ref[r, :] for r in range(6)]
    v2f = _transform(g, f2vp, w_ref[0, 0], w_ref[0, 1], w_ref[1, 0], w_ref[1, 1])
    fb = [fp[s] + v2f[2 * 0 + (s >> 2)] + v2f[2 * 1 + ((s >> 1) & 1)]
          + v2f[2 * 2 + (s & 1)] for s in range(8)]
    l8 = _lse2(_lse2(_lse2(fb[0], fb[1]), _lse2(fb[2], fb[3])),
               _lse2(_lse2(fb[4], fb[5]), _lse2(fb[6], fb[7])))
    pos = pl.program_id(0) * BF + lax.broadcasted_iota(jnp.int32, fp[0].shape, 0)
    mask = (pos < F).astype(jnp.float32)
    part = jnp.zeros_like(fp[0])
    for s in range(8):
        fbn = fb[s] - l8
        e = jnp.exp(fbn)
        part = part + e * (fp[s] - fbn)
    part = jnp.sum(part * mask)

    @pl.when(pl.program_id(0) == 0)
    def _():
        out_ref[0, 0] = 0.0

    out_ref[0, 0] += part


def _kg_body(vbn_ref, degp_ref, out_ref):
    vb0 = vbn_ref[0, :]
    vb1 = vbn_ref[1, :]
    deg = degp_ref[0, :] + degp_ref[1, :]
    inner = jnp.exp(vb0) * vb0 + jnp.exp(vb1) * vb1
    pos = pl.program_id(0) * BN + lax.broadcasted_iota(jnp.int32, vb0.shape, 0)
    mask = (pos < N).astype(jnp.float32)
    part = jnp.sum((deg - 1.0) * inner * mask)

    @pl.when(pl.program_id(0) == 0)
    def _():
        out_ref[0, 0] = 0.0

    out_ref[0, 0] += part


_GF = F_PAD // BF
_GN = N_PAD // BN

_ka0 = pl.pallas_call(
    _ka0_body,
    grid=(_GF,),
    in_specs=[pl.BlockSpec((8, BF), lambda i: (0, i))],
    out_specs=pl.BlockSpec((8, BF), lambda i: (0, i)),
    out_shape=jax.ShapeDtypeStruct((8, F_PAD), jnp.float32),
)

_ka = pl.pallas_call(
    _ka_body,
    grid=(_GF,),
    in_specs=[
        pl.BlockSpec((8, BF), lambda i: (0, i)),
        pl.BlockSpec((8, BF), lambda i: (0, i)),
        pl.BlockSpec((8, BF), lambda i: (0, i)),
        pl.BlockSpec(memory_space=pltpu.SMEM),
    ],
    out_specs=pl.BlockSpec((8, BF), lambda i: (0, i)),
    out_shape=jax.ShapeDtypeStruct((8, F_PAD), jnp.float32),
)

_kc = pl.pallas_call(
    _kc_body,
    grid=(_GN,),
    in_specs=[pl.BlockSpec((4, BN), lambda i: (0, i))],
    out_specs=pl.BlockSpec((2, BN), lambda i: (0, i)),
    out_shape=jax.ShapeDtypeStruct((2, N_PAD), jnp.float32),
)

_kf = pl.pallas_call(
    _kf_body,
    grid=(_GF,),
    in_specs=[
        pl.BlockSpec((8, BF), lambda i: (0, i)),
        pl.BlockSpec((8, BF), lambda i: (0, i)),
        pl.BlockSpec((8, BF), lambda i: (0, i)),
        pl.BlockSpec(memory_space=pltpu.SMEM),
    ],
    out_specs=pl.BlockSpec((1, 1), lambda i: (0, 0), memory_space=pltpu.SMEM),
    out_shape=jax.ShapeDtypeStruct((1, 1), jnp.float32),
)

_kg = pl.pallas_call(
    _kg_body,
    grid=(_GN,),
    in_specs=[
        pl.BlockSpec((2, BN), lambda i: (0, i)),
        pl.BlockSpec((2, BN), lambda i: (0, i)),
    ],
    out_specs=pl.BlockSpec((1, 1), lambda i: (0, 0), memory_space=pltpu.SMEM),
    out_shape=jax.ShapeDtypeStruct((1, 1), jnp.float32),
)


def _scatter(f2v, idx_t):
    vb0 = jnp.zeros((N_PAD,), jnp.float32)
    vb1 = jnp.zeros((N_PAD,), jnp.float32)
    for d in range(3):
        vb0 = vb0.at[idx_t[d]].add(f2v[2 * d])
        vb1 = vb1.at[idx_t[d]].add(f2v[2 * d + 1])
    return jnp.stack([vb0, vb1, jnp.zeros_like(vb0), jnp.zeros_like(vb1)])


def _gather(vbn, idx_t):
    rows = []
    for d in range(3):
        for x in range(2):
            rows.append(vbn[x][idx_t[d]])
    rows += [jnp.zeros_like(rows[0])] * 2
    return jnp.stack(rows)


def _degrees(idx_t, ones_mask):
    deg = jnp.zeros((N_PAD,), jnp.float32)
    for d in range(3):
        deg = deg.at[idx_t[d]].add(ones_mask)
    return jnp.stack([deg, jnp.zeros_like(deg)])


def kernel(factor_potentials, edge_var_indices, layer_weights):
    fp_t = jnp.pad(factor_potentials.reshape(F, 8).T, ((0, 0), (0, F_PAD - F)))
    idx_t = jnp.pad(edge_var_indices.reshape(F, 3).T, ((0, 0), (0, F_PAD - F)))
    ones_mask = (jnp.arange(F_PAD) < F).astype(jnp.float32)

    degp = _degrees(idx_t, ones_mask)

    f2v = _ka0(fp_t)
    vbn = None
    g = None
    for it in range(ITERS):
        if it > 0:
            f2v = _ka(fp_t, g, f2v, layer_weights[it - 1])
        vbp = _scatter(f2v, idx_t)
        vbn = _kc(vbp)
        g = _gather(vbn, idx_t)
    s_f = _kf(fp_t, g, f2v, layer_weights[ITERS - 1])
    s_n = _kg(vbn, degp)
    return (s_f[0, 0] + s_n[0, 0]).astype(jnp.float32)


# speedup vs baseline: 12.1034x; 6.1792x over previous
import functools

import jax
import jax.numpy as jnp
from jax import lax
from jax.experimental import pallas as pl
from jax.experimental.pallas import tpu as pltpu
from jax.experimental.pallas import tpu_sc as plsc

N = 100000
F = 200000
D = 3
ITERS = 5

NC = 2
NS = 16
F_PAD = 200704
N_PAD = 100352
BF = 512
BN = 512
CHE = F_PAD // (NC * NS)
CHN = N_PAD // NS


def _lse2(a, b):
    m = jnp.maximum(a, b)
    return m + jnp.log1p(jnp.exp(-jnp.abs(a - b)))


def _factor_messages(fp, v2f):
    fb = [fp[s] + v2f[2 * 0 + (s >> 2)] + v2f[2 * 1 + ((s >> 1) & 1)]
          + v2f[2 * 2 + (s & 1)] for s in range(8)]
    out = []
    for d in range(3):
        msgs = []
        for x in range(2):
            rows = [fb[s] for s in range(8) if ((s >> (2 - d)) & 1) == x]
            marg = _lse2(_lse2(rows[0], rows[1]), _lse2(rows[2], rows[3]))
            msgs.append(marg - v2f[2 * d + x])
        l = _lse2(msgs[0], msgs[1])
        out.append(msgs[0] - l)
        out.append(msgs[1] - l)
    return out


def _transform(g, f2v, w00, w01, w10, w11):
    out = []
    for d in range(3):
        u0 = g[2 * d] - f2v[2 * d]
        u1 = g[2 * d + 1] - f2v[2 * d + 1]
        v0 = u0 + u0 * w00 + u1 * w10
        v1 = u1 + u0 * w01 + u1 * w11
        l = _lse2(v0, v1)
        out.append(v0 - l)
        out.append(v1 - l)
    return out


def _ka0_body(fp_ref, f2v_ref):
    fp = fp_ref[:, :]
    zero = jnp.zeros_like(fp[0])
    v2f = [zero] * 6
    f2v = _factor_messages(fp, v2f)
    f2v_ref[:, :] = jnp.stack(f2v + [zero, zero])


def _ka_body(fp_ref, g_ref, f2vp_ref, w_ref, f2v_ref):
    fp = fp_ref[:, :]
    g = [g_ref[r, :] for r in range(6)]
    f2vp = [f2vp_ref[r, :] for r in range(6)]
    v2f = _transform(g, f2vp, w_ref[0, 0], w_ref[0, 1], w_ref[1, 0], w_ref[1, 1])
    f2v = _factor_messages(fp, v2f)
    zero = jnp.zeros_like(fp[0])
    f2v_ref[:, :] = jnp.stack(f2v + [zero, zero])


def _kc_body(p0_ref, p1_ref, p2_ref, p3_ref, n0_ref, n1_ref):
    vb0 = p0_ref[...] + p2_ref[...]
    vb1 = p1_ref[...] + p3_ref[...]
    l = _lse2(vb0, vb1)
    n0_ref[...] = vb0 - l
    n1_ref[...] = vb1 - l


def _kf_body(fp_ref, g_ref, f2vp_ref, w_ref, out_ref):
    fp = fp_ref[:, :]
    g = [g_ref[r, :] for r in range(6)]
    f2vp = [f2vp_ref[r, :] for r in range(6)]
    v2f = _transform(g, f2vp, w_ref[0, 0], w_ref[0, 1], w_ref[1, 0], w_ref[1, 1])
    fb = [fp[s] + v2f[2 * 0 + (s >> 2)] + v2f[2 * 1 + ((s >> 1) & 1)]
          + v2f[2 * 2 + (s & 1)] for s in range(8)]
    l8 = _lse2(_lse2(_lse2(fb[0], fb[1]), _lse2(fb[2], fb[3])),
               _lse2(_lse2(fb[4], fb[5]), _lse2(fb[6], fb[7])))
    pos = pl.program_id(0) * BF + lax.broadcasted_iota(jnp.int32, fp[0].shape, 0)
    mask = (pos < F).astype(jnp.float32)
    part = jnp.zeros_like(fp[0])
    for s in range(8):
        fbn = fb[s] - l8
        e = jnp.exp(fbn)
        part = part + e * (fp[s] - fbn)
    part = jnp.sum(part * mask)

    @pl.when(pl.program_id(0) == 0)
    def _():
        out_ref[0, 0] = 0.0

    out_ref[0, 0] += part


def _kg_body(n0_ref, n1_ref, d0_ref, d1_ref, out_ref):
    vb0 = n0_ref[...]
    vb1 = n1_ref[...]
    deg = d0_ref[...] + d1_ref[...]
    inner = jnp.exp(vb0) * vb0 + jnp.exp(vb1) * vb1
    pos = pl.program_id(0) * BN + lax.broadcasted_iota(jnp.int32, vb0.shape, 0)
    mask = (pos < N).astype(jnp.float32)
    part = jnp.sum((deg - 1.0) * inner * mask)

    @pl.when(pl.program_id(0) == 0)
    def _():
        out_ref[0, 0] = 0.0

    out_ref[0, 0] += part


_GF = F_PAD // BF
_GN = N_PAD // BN

_spec8 = pl.BlockSpec((8, BF), lambda i: (0, i))
_spec1n = pl.BlockSpec((BN,), lambda i: (i,))

_ka0 = pl.pallas_call(
    _ka0_body,
    grid=(_GF,),
    in_specs=[_spec8],
    out_specs=_spec8,
    out_shape=jax.ShapeDtypeStruct((8, F_PAD), jnp.float32),
)

_ka = pl.pallas_call(
    _ka_body,
    grid=(_GF,),
    in_specs=[_spec8, _spec8, _spec8, pl.BlockSpec(memory_space=pltpu.SMEM)],
    out_specs=_spec8,
    out_shape=jax.ShapeDtypeStruct((8, F_PAD), jnp.float32),
)

_kc = pl.pallas_call(
    _kc_body,
    grid=(_GN,),
    in_specs=[_spec1n, _spec1n, _spec1n, _spec1n],
    out_specs=(_spec1n, _spec1n),
    out_shape=(jax.ShapeDtypeStruct((N_PAD,), jnp.float32),
               jax.ShapeDtypeStruct((N_PAD,), jnp.float32)),
)

_kf = pl.pallas_call(
    _kf_body,
    grid=(_GF,),
    in_specs=[_spec8, _spec8, _spec8, pl.BlockSpec(memory_space=pltpu.SMEM)],
    out_specs=pl.BlockSpec((1, 1), lambda i: (0, 0), memory_space=pltpu.SMEM),
    out_shape=jax.ShapeDtypeStruct((1, 1), jnp.float32),
)

_kg = pl.pallas_call(
    _kg_body,
    grid=(_GN,),
    in_specs=[_spec1n, _spec1n, _spec1n, _spec1n],
    out_specs=pl.BlockSpec((1, 1), lambda i: (0, 0), memory_space=pltpu.SMEM),
    out_shape=jax.ShapeDtypeStruct((1, 1), jnp.float32),
)


_sc_mesh = plsc.VectorSubcoreMesh(core_axis_name="c", subcore_axis_name="s")


def _zero_fill(buf, n):
    def body(i, carry):
        buf[pl.ds(i * 16, 16)] = jnp.zeros((16,), jnp.float32)
        return carry
    lax.fori_loop(0, n // 16, body, 0, unroll=4)


def _flatten_row(src2d, r, dst1d, n):
    def body(i, carry):
        dst1d[pl.ds(i * 16, 16)] = src2d[r, pl.ds(i * 16, 16)]
        return carry
    lax.fori_loop(0, n // 16, body, 0, unroll=4)


@functools.partial(
    pl.kernel, mesh=_sc_mesh,
    out_type=tuple(jax.ShapeDtypeStruct((N_PAD,), jnp.float32)
                   for _ in range(4)),
    scratch_types=[
        pltpu.VMEM((8, CHE), jnp.float32),
        pltpu.VMEM((CHE,), jnp.int32),
        pltpu.VMEM((CHE,), jnp.float32),
        pltpu.VMEM((CHE,), jnp.float32),
        pltpu.VMEM((CHN,), jnp.float32),
        pltpu.VMEM_SHARED((N_PAD,), jnp.float32),
        pltpu.VMEM_SHARED((N_PAD,), jnp.float32),
    ],
)
def _scatter(f2v_hbm, idx0_hbm, idx1_hbm, idx2_hbm,
             o0_hbm, o1_hbm, o2_hbm, o3_hbm,
             fbuf, idx_v, v0, v1, zbuf, acc0, acc1):
    c = lax.axis_index("c")
    s = lax.axis_index("s")
    w = c * NS + s
    _zero_fill(zbuf, CHN)
    pltpu.sync_copy(zbuf, acc0.at[pl.ds(s * CHN, CHN)])
    pltpu.sync_copy(zbuf, acc1.at[pl.ds(s * CHN, CHN)])
    plsc.subcore_barrier()
    base = w * CHE
    pltpu.sync_copy(f2v_hbm.at[:, pl.ds(base, CHE)], fbuf)
    for d, idx_hbm in enumerate((idx0_hbm, idx1_hbm, idx2_hbm)):
        pltpu.sync_copy(idx_hbm.at[pl.ds(base, CHE)], idx_v)
        _flatten_row(fbuf, 2 * d, v0, CHE)
        _flatten_row(fbuf, 2 * d + 1, v1, CHE)
        pltpu.sync_copy(v0, acc0.at[idx_v], add=True)
        pltpu.sync_copy(v1, acc1.at[idx_v], add=True)
    plsc.subcore_barrier()

    @pl.when(c == 0)
    def _():
        pltpu.sync_copy(acc0.at[pl.ds(s * CHN, CHN)], o0_hbm.at[pl.ds(s * CHN, CHN)])
        pltpu.sync_copy(acc1.at[pl.ds(s * CHN, CHN)], o1_hbm.at[pl.ds(s * CHN, CHN)])

    @pl.when(c == 1)
    def _():
        pltpu.sync_copy(acc0.at[pl.ds(s * CHN, CHN)], o2_hbm.at[pl.ds(s * CHN, CHN)])
        pltpu.sync_copy(acc1.at[pl.ds(s * CHN, CHN)], o3_hbm.at[pl.ds(s * CHN, CHN)])


@functools.partial(
    pl.kernel, mesh=_sc_mesh,
    out_type=jax.ShapeDtypeStruct((8, F_PAD), jnp.float32),
    scratch_types=[
        pltpu.VMEM((8, CHE), jnp.float32),
        pltpu.VMEM((CHE,), jnp.int32),
        pltpu.VMEM((CHE,), jnp.float32),
        pltpu.VMEM((CHN,), jnp.float32),
        pltpu.VMEM_SHARED((N_PAD,), jnp.float32),
        pltpu.VMEM_SHARED((N_PAD,), jnp.float32),
        pltpu.SemaphoreType.DMA,
    ],
)
def _gather(n0_hbm, n1_hbm, idx0_hbm, idx1_hbm, idx2_hbm, g_hbm,
            gbuf, idx_v, g1, tmp, sh0, sh1, sem):
    c = lax.axis_index("c")
    s = lax.axis_index("s")
    w = c * NS + s
    pltpu.sync_copy(n0_hbm.at[pl.ds(s * CHN, CHN)], tmp)
    pltpu.sync_copy(tmp, sh0.at[pl.ds(s * CHN, CHN)])
    pltpu.sync_copy(n1_hbm.at[pl.ds(s * CHN, CHN)], tmp)
    pltpu.sync_copy(tmp, sh1.at[pl.ds(s * CHN, CHN)])
    def zrow(i, carry):
        z = jnp.zeros((16,), jnp.float32)
        gbuf[6, pl.ds(i * 16, 16)] = z
        gbuf[7, pl.ds(i * 16, 16)] = z
        return carry
    lax.fori_loop(0, CHE // 16, zrow, 0, unroll=4)
    plsc.subcore_barrier()
    base = w * CHE
    for d, idx_hbm in enumerate((idx0_hbm, idx1_hbm, idx2_hbm)):
        pltpu.sync_copy(idx_hbm.at[pl.ds(base, CHE)], idx_v)
        pltpu.async_copy(sh0.at[idx_v], g1, sem).wait()

        def cp0(i, carry):
            gbuf[2 * d, pl.ds(i * 16, 16)] = g1[pl.ds(i * 16, 16)]
            return carry
        lax.fori_loop(0, CHE // 16, cp0, 0, unroll=4)
        pltpu.async_copy(sh1.at[idx_v], g1, sem).wait()

        def cp1(i, carry):
            gbuf[2 * d + 1, pl.ds(i * 16, 16)] = g1[pl.ds(i * 16, 16)]
            return carry
        lax.fori_loop(0, CHE // 16, cp1, 0, unroll=4)
    pltpu.sync_copy(gbuf, g_hbm.at[:, pl.ds(base, CHE)])


@functools.partial(
    pl.kernel, mesh=_sc_mesh,
    out_type=(jax.ShapeDtypeStruct((N_PAD,), jnp.float32),
              jax.ShapeDtypeStruct((N_PAD,), jnp.float32)),
    scratch_types=[
        pltpu.VMEM((CHE,), jnp.int32),
        pltpu.VMEM((CHE,), jnp.float32),
        pltpu.VMEM((CHN,), jnp.float32),
        pltpu.VMEM_SHARED((N_PAD,), jnp.float32),
    ],
)
def _degrees(idx0_hbm, idx1_hbm, idx2_hbm, ones_hbm, d0_hbm, d1_hbm,
             idx_v, val_v, zbuf, acc0):
    c = lax.axis_index("c")
    s = lax.axis_index("s")
    w = c * NS + s
    _zero_fill(zbuf, CHN)
    pltpu.sync_copy(zbuf, acc0.at[pl.ds(s * CHN, CHN)])
    plsc.subcore_barrier()
    base = w * CHE
    pltpu.sync_copy(ones_hbm.at[pl.ds(base, CHE)], val_v)
    for idx_hbm in (idx0_hbm, idx1_hbm, idx2_hbm):
        pltpu.sync_copy(idx_hbm.at[pl.ds(base, CHE)], idx_v)
        pltpu.sync_copy(val_v, acc0.at[idx_v], add=True)
    plsc.subcore_barrier()

    @pl.when(c == 0)
    def _():
        pltpu.sync_copy(acc0.at[pl.ds(s * CHN, CHN)], d0_hbm.at[pl.ds(s * CHN, CHN)])

    @pl.when(c == 1)
    def _():
        pltpu.sync_copy(acc0.at[pl.ds(s * CHN, CHN)], d1_hbm.at[pl.ds(s * CHN, CHN)])


def kernel(factor_potentials, edge_var_indices, layer_weights):
    fp_t = jnp.pad(factor_potentials.reshape(F, 8).T, ((0, 0), (0, F_PAD - F)))
    idx3 = edge_var_indices.reshape(F, 3).T
    pad = F_PAD - F
    idx0 = jnp.pad(idx3[0], (0, pad))
    idx1 = jnp.pad(idx3[1], (0, pad))
    idx2 = jnp.pad(idx3[2], (0, pad))
    ones_mask = (jnp.arange(F_PAD) < F).astype(jnp.float32)

    deg0, deg1 = _degrees(idx0, idx1, idx2, ones_mask)

    f2v = _ka0(fp_t)
    vbn = None
    g = None
    for it in range(ITERS):
        if it > 0:
            f2v = _ka(fp_t, g, f2v, layer_weights[it - 1])
        vbp = _scatter(f2v, idx0, idx1, idx2)
        vbn = _kc(*vbp)
        g = _gather(vbn[0], vbn[1], idx0, idx1, idx2)
    s_f = _kf(fp_t, g, f2v, layer_weights[ITERS - 1])
    s_n = _kg(vbn[0], vbn[1], deg0, deg1)
    return (s_f[0, 0] + s_n[0, 0]).astype(jnp.float32)


# speedup vs baseline: 30.9214x; 2.5548x over previous
import functools

import jax
import jax.numpy as jnp
from jax import lax
from jax.experimental import pallas as pl
from jax.experimental.pallas import tpu as pltpu
from jax.experimental.pallas import tpu_sc as plsc

N = 100000
F = 200000
D = 3
ITERS = 5

NC = 2
NS = 16
F_PAD = 200704
N_PAD = 102400
NR = 200
BF = 4096
CHE = F_PAD // (NC * NS)
CHN = N_PAD // NS


def _rowi():
    return lax.broadcasted_iota(jnp.int32, (8, 1), 0)


def _mk_mask67():
    return (_rowi() < 6).astype(jnp.float32)


def _bc(row):
    return jnp.broadcast_to(row, (8, row.shape[1]))


def _pair_normalize(v):
    even = (_rowi() % 2) == 0
    sw = jnp.where(even, jnp.roll(v, -1, axis=0), jnp.roll(v, 1, axis=0))
    m = jnp.maximum(v, sw)
    l = m + jnp.log1p(jnp.exp(-jnp.abs(v - sw)))
    return v - l


def _expand_msgs(v2f):
    r = _rowi()
    v0 = jnp.where((r >> 2) & 1 == 0, _bc(v2f[0:1]), _bc(v2f[1:2]))
    v1 = jnp.where((r >> 1) & 1 == 0, _bc(v2f[2:3]), _bc(v2f[3:4]))
    v2 = jnp.where(r & 1 == 0, _bc(v2f[4:5]), _bc(v2f[5:6]))
    return v0 + v1 + v2


def _factor_messages(fp, v2f):
    fb = fp + _expand_msgs(v2f)
    m = jnp.max(fb, axis=0, keepdims=True)
    e = jnp.exp(fb - m)
    s8 = jnp.sum(e, axis=0, keepdims=True)
    r = _rowi()
    t0 = jnp.sum(jnp.where((r >> 2) & 1 == 0, e, 0.0), axis=0, keepdims=True)
    t1 = jnp.sum(jnp.where((r >> 1) & 1 == 0, e, 0.0), axis=0, keepdims=True)
    t2 = jnp.sum(jnp.where(r & 1 == 0, e, 0.0), axis=0, keepdims=True)
    z = jnp.concatenate([t0, s8 - t0, t1, s8 - t1, t2, s8 - t2, s8, s8], axis=0)
    marg = m + jnp.log(z)
    msg = marg - v2f
    return _pair_normalize(msg) * _mk_mask67()


def _transform(g, f2vp, w_ref):
    u = g - f2vp
    r = _rowi()
    even = (r % 2) == 0
    valid = (r < 6).astype(jnp.float32)
    ue = jnp.where(even, u, jnp.roll(u, 1, axis=0))
    uo = jnp.where(even, jnp.roll(u, -1, axis=0), u)
    ce = jnp.where(even, w_ref[0, 0], w_ref[0, 1]) * valid
    co = jnp.where(even, w_ref[1, 0], w_ref[1, 1]) * valid
    v = u + ue * ce + uo * co
    return _pair_normalize(v) * _mk_mask67()


def _ka0_body(fp_ref, f2v_ref):
    fp = fp_ref[:, :]
    f2v_ref[:, :] = _factor_messages(fp, jnp.zeros_like(fp))


def _ka_body(fp_ref, g_ref, f2vp_ref, w_ref, f2v_ref):
    v2f = _transform(g_ref[:, :], f2vp_ref[:, :], w_ref)
    f2v_ref[:, :] = _factor_messages(fp_ref[:, :], v2f)


def _kc_body(p0_ref, p1_ref, p2_ref, p3_ref, n0_ref, n1_ref):
    vb0 = p0_ref[:, :] + p2_ref[:, :]
    vb1 = p1_ref[:, :] + p3_ref[:, :]
    m = jnp.maximum(vb0, vb1)
    l = m + jnp.log1p(jnp.exp(-jnp.abs(vb0 - vb1)))
    n0_ref[:, :] = vb0 - l
    n1_ref[:, :] = vb1 - l


def _kf_body(fp_ref, g_ref, f2vp_ref, w_ref, out_ref):
    fp = fp_ref[:, :]
    v2f = _transform(g_ref[:, :], f2vp_ref[:, :], w_ref)
    fb = fp + _expand_msgs(v2f)
    m = jnp.max(fb, axis=0, keepdims=True)
    e = jnp.exp(fb - m)
    l8 = m + jnp.log(jnp.sum(e, axis=0, keepdims=True))
    e8 = jnp.exp(fb - l8)
    t = e8 * (fp - fb + l8)
    pos = pl.program_id(0) * BF + lax.broadcasted_iota(jnp.int32, (8, BF), 1)
    part = jnp.sum(jnp.where(pos < F, t, 0.0))

    @pl.when(pl.program_id(0) == 0)
    def _():
        out_ref[0, 0] = 0.0

    out_ref[0, 0] += part


def _kg_body(n0_ref, n1_ref, d0_ref, d1_ref, out_ref):
    vb0 = n0_ref[:, :]
    vb1 = n1_ref[:, :]
    deg = d0_ref[:, :] + d1_ref[:, :]
    inner = jnp.exp(vb0) * vb0 + jnp.exp(vb1) * vb1
    rows = pl.program_id(0) * 8 + lax.broadcasted_iota(jnp.int32, (8, 512), 0)
    pos = rows * 512 + lax.broadcasted_iota(jnp.int32, (8, 512), 1)
    part = jnp.sum(jnp.where(pos < N, (deg - 1.0) * inner, 0.0))

    @pl.when(pl.program_id(0) == 0)
    def _():
        out_ref[0, 0] = 0.0

    out_ref[0, 0] += part


_GF = F_PAD // BF
_GN = NR // 8

_spec8 = pl.BlockSpec((8, BF), lambda i: (0, i))
_specn = pl.BlockSpec((8, 512), lambda i: (i, 0))
_spec11 = pl.BlockSpec((1, 1), lambda i: (0, 0), memory_space=pltpu.SMEM)

_ka0 = pl.pallas_call(
    _ka0_body,
    grid=(_GF,),
    in_specs=[_spec8],
    out_specs=_spec8,
    out_shape=jax.ShapeDtypeStruct((8, F_PAD), jnp.float32),
)

_ka = pl.pallas_call(
    _ka_body,
    grid=(_GF,),
    in_specs=[_spec8, _spec8, _spec8, pl.BlockSpec(memory_space=pltpu.SMEM)],
    out_specs=_spec8,
    out_shape=jax.ShapeDtypeStruct((8, F_PAD), jnp.float32),
)

_kc = pl.pallas_call(
    _kc_body,
    grid=(_GN,),
    in_specs=[_specn, _specn, _specn, _specn],
    out_specs=(_specn, _specn),
    out_shape=(jax.ShapeDtypeStruct((NR, 512), jnp.float32),
               jax.ShapeDtypeStruct((NR, 512), jnp.float32)),
)

_kf = pl.pallas_call(
    _kf_body,
    grid=(_GF,),
    in_specs=[_spec8, _spec8, _spec8, pl.BlockSpec(memory_space=pltpu.SMEM)],
    out_specs=_spec11,
    out_shape=jax.ShapeDtypeStruct((1, 1), jnp.float32),
)

_kg = pl.pallas_call(
    _kg_body,
    grid=(_GN,),
    in_specs=[_specn, _specn, _specn, _specn],
    out_specs=_spec11,
    out_shape=jax.ShapeDtypeStruct((1, 1), jnp.float32),
)


_sc_mesh = plsc.VectorSubcoreMesh(core_axis_name="c", subcore_axis_name="s")


def _zero_fill(buf, n):
    def body(i, carry):
        buf[pl.ds(i * 16, 16)] = jnp.zeros((16,), jnp.float32)
        return carry
    lax.fori_loop(0, n // 16, body, 0, unroll=4)


def _flatten_row(src2d, r, dst1d, n):
    def body(i, carry):
        dst1d[pl.ds(i * 16, 16)] = src2d[r, pl.ds(i * 16, 16)]
        return carry
    lax.fori_loop(0, n // 16, body, 0, unroll=4)


@functools.partial(
    pl.kernel, mesh=_sc_mesh,
    out_type=tuple(jax.ShapeDtypeStruct((N_PAD,), jnp.float32)
                   for _ in range(4)),
    scratch_types=[
        pltpu.VMEM((8, CHE), jnp.float32),
        pltpu.VMEM((CHE,), jnp.int32),
        pltpu.VMEM((CHE,), jnp.float32),
        pltpu.VMEM((CHE,), jnp.float32),
        pltpu.VMEM((CHN,), jnp.float32),
        pltpu.VMEM_SHARED((N_PAD,), jnp.float32),
        pltpu.VMEM_SHARED((N_PAD,), jnp.float32),
    ],
)
def _scatter(f2v_hbm, idx0_hbm, idx1_hbm, idx2_hbm,
             o0_hbm, o1_hbm, o2_hbm, o3_hbm,
             fbuf, idx_v, v0, v1, zbuf, acc0, acc1):
    c = lax.axis_index("c")
    s = lax.axis_index("s")
    w = c * NS + s
    _zero_fill(zbuf, CHN)
    pltpu.sync_copy(zbuf, acc0.at[pl.ds(s * CHN, CHN)])
    pltpu.sync_copy(zbuf, acc1.at[pl.ds(s * CHN, CHN)])
    plsc.subcore_barrier()
    base = w * CHE
    pltpu.sync_copy(f2v_hbm.at[:, pl.ds(base, CHE)], fbuf)
    for d, idx_hbm in enumerate((idx0_hbm, idx1_hbm, idx2_hbm)):
        pltpu.sync_copy(idx_hbm.at[pl.ds(base, CHE)], idx_v)
        _flatten_row(fbuf, 2 * d, v0, CHE)
        _flatten_row(fbuf, 2 * d + 1, v1, CHE)
        pltpu.sync_copy(v0, acc0.at[idx_v], add=True)
        pltpu.sync_copy(v1, acc1.at[idx_v], add=True)
    plsc.subcore_barrier()

    @pl.when(c == 0)
    def _():
        pltpu.sync_copy(acc0.at[pl.ds(s * CHN, CHN)], o0_hbm.at[pl.ds(s * CHN, CHN)])
        pltpu.sync_copy(acc1.at[pl.ds(s * CHN, CHN)], o1_hbm.at[pl.ds(s * CHN, CHN)])

    @pl.when(c == 1)
    def _():
        pltpu.sync_copy(acc0.at[pl.ds(s * CHN, CHN)], o2_hbm.at[pl.ds(s * CHN, CHN)])
        pltpu.sync_copy(acc1.at[pl.ds(s * CHN, CHN)], o3_hbm.at[pl.ds(s * CHN, CHN)])


@functools.partial(
    pl.kernel, mesh=_sc_mesh,
    out_type=jax.ShapeDtypeStruct((8, F_PAD), jnp.float32),
    scratch_types=[
        pltpu.VMEM((8, CHE), jnp.float32),
        pltpu.VMEM((CHE,), jnp.int32),
        pltpu.VMEM((CHE,), jnp.float32),
        pltpu.VMEM((CHN,), jnp.float32),
        pltpu.VMEM_SHARED((N_PAD,), jnp.float32),
        pltpu.VMEM_SHARED((N_PAD,), jnp.float32),
        pltpu.SemaphoreType.DMA,
    ],
)
def _gather(n0_hbm, n1_hbm, idx0_hbm, idx1_hbm, idx2_hbm, g_hbm,
            gbuf, idx_v, g1, tmp, sh0, sh1, sem):
    c = lax.axis_index("c")
    s = lax.axis_index("s")
    w = c * NS + s
    pltpu.sync_copy(n0_hbm.at[pl.ds(s * CHN, CHN)], tmp)
    pltpu.sync_copy(tmp, sh0.at[pl.ds(s * CHN, CHN)])
    pltpu.sync_copy(n1_hbm.at[pl.ds(s * CHN, CHN)], tmp)
    pltpu.sync_copy(tmp, sh1.at[pl.ds(s * CHN, CHN)])
    def zrow(i, carry):
        z = jnp.zeros((16,), jnp.float32)
        gbuf[6, pl.ds(i * 16, 16)] = z
        gbuf[7, pl.ds(i * 16, 16)] = z
        return carry
    lax.fori_loop(0, CHE // 16, zrow, 0, unroll=4)
    plsc.subcore_barrier()
    base = w * CHE
    for d, idx_hbm in enumerate((idx0_hbm, idx1_hbm, idx2_hbm)):
        pltpu.sync_copy(idx_hbm.at[pl.ds(base, CHE)], idx_v)
        pltpu.async_copy(sh0.at[idx_v], g1, sem).wait()

        def cp0(i, carry):
            gbuf[2 * d, pl.ds(i * 16, 16)] = g1[pl.ds(i * 16, 16)]
            return carry
        lax.fori_loop(0, CHE // 16, cp0, 0, unroll=4)
        pltpu.async_copy(sh1.at[idx_v], g1, sem).wait()

        def cp1(i, carry):
            gbuf[2 * d + 1, pl.ds(i * 16, 16)] = g1[pl.ds(i * 16, 16)]
            return carry
        lax.fori_loop(0, CHE // 16, cp1, 0, unroll=4)
    pltpu.sync_copy(gbuf, g_hbm.at[:, pl.ds(base, CHE)])


@functools.partial(
    pl.kernel, mesh=_sc_mesh,
    out_type=(jax.ShapeDtypeStruct((N_PAD,), jnp.float32),
              jax.ShapeDtypeStruct((N_PAD,), jnp.float32)),
    scratch_types=[
        pltpu.VMEM((CHE,), jnp.int32),
        pltpu.VMEM((CHE,), jnp.float32),
        pltpu.VMEM((CHN,), jnp.float32),
        pltpu.VMEM_SHARED((N_PAD,), jnp.float32),
    ],
)
def _degrees(idx0_hbm, idx1_hbm, idx2_hbm, ones_hbm, d0_hbm, d1_hbm,
             idx_v, val_v, zbuf, acc0):
    c = lax.axis_index("c")
    s = lax.axis_index("s")
    w = c * NS + s
    _zero_fill(zbuf, CHN)
    pltpu.sync_copy(zbuf, acc0.at[pl.ds(s * CHN, CHN)])
    plsc.subcore_barrier()
    base = w * CHE
    pltpu.sync_copy(ones_hbm.at[pl.ds(base, CHE)], val_v)
    for idx_hbm in (idx0_hbm, idx1_hbm, idx2_hbm):
        pltpu.sync_copy(idx_hbm.at[pl.ds(base, CHE)], idx_v)
        pltpu.sync_copy(val_v, acc0.at[idx_v], add=True)
    plsc.subcore_barrier()

    @pl.when(c == 0)
    def _():
        pltpu.sync_copy(acc0.at[pl.ds(s * CHN, CHN)], d0_hbm.at[pl.ds(s * CHN, CHN)])

    @pl.when(c == 1)
    def _():
        pltpu.sync_copy(acc0.at[pl.ds(s * CHN, CHN)], d1_hbm.at[pl.ds(s * CHN, CHN)])


def kernel(factor_potentials, edge_var_indices, layer_weights):
    fp_t = jnp.pad(factor_potentials.reshape(F, 8).T, ((0, 0), (0, F_PAD - F)))
    idx3 = edge_var_indices.reshape(F, 3).T
    pad = F_PAD - F
    idx0 = jnp.pad(idx3[0], (0, pad))
    idx1 = jnp.pad(idx3[1], (0, pad))
    idx2 = jnp.pad(idx3[2], (0, pad))
    ones_mask = (jnp.arange(F_PAD) < F).astype(jnp.float32)

    deg0, deg1 = _degrees(idx0, idx1, idx2, ones_mask)
    deg0_2d = deg0.reshape(NR, 512)
    deg1_2d = deg1.reshape(NR, 512)

    f2v = _ka0(fp_t)
    vbn = None
    g = None
    for it in range(ITERS):
        if it > 0:
            f2v = _ka(fp_t, g, f2v, layer_weights[it - 1])
        vbp = _scatter(f2v, idx0, idx1, idx2)
        vbn = _kc(*(p.reshape(NR, 512) for p in vbp))
        g = _gather(vbn[0].reshape(N_PAD), vbn[1].reshape(N_PAD),
                    idx0, idx1, idx2)
    s_f = _kf(fp_t, g, f2v, layer_weights[ITERS - 1])
    s_n = _kg(vbn[0], vbn[1], deg0_2d, deg1_2d)
    return (s_f[0, 0] + s_n[0, 0]).astype(jnp.float32)
